# initial kernel scaffold (unmeasured)
import jax
import jax.numpy as jnp
from jax import lax
from jax.experimental import pallas as pl
from jax.experimental.pallas import tpu as pltpu

N_DEV = 16


def kernel(x, w_mat):
    k_dim, k_shard = x.shape
    n = w_mat.shape[1]
    m_blk = k_dim // N_DEV

    def body(x_ref, w_ref, out_ref, gather_ref, y_ref, amax_ref,
             send_sems, recv_sems, send_sems2, recv_sems2):
        me = lax.axis_index("i")

        barrier_sem = pltpu.get_barrier_semaphore()
        for d in range(N_DEV):
            @pl.when(me != d)
            def _():
                pl.semaphore_signal(
                    barrier_sem, inc=1,
                    device_id=(d,), device_id_type=pl.DeviceIdType.MESH,
                )
        pl.semaphore_wait(barrier_sem, N_DEV - 1)

        for j in range(N_DEV):
            @pl.when(me != j)
            def _():
                rdma = pltpu.make_async_remote_copy(
                    src_ref=x_ref.at[pl.ds(j * m_blk, m_blk), :],
                    dst_ref=gather_ref.at[:, pl.ds(me * k_shard, k_shard)],
                    send_sem=send_sems.at[j],
                    recv_sem=recv_sems.at[me],
                    device_id=(j,),
                    device_id_type=pl.DeviceIdType.MESH,
                )
                rdma.start()

        @pl.when(me == me)
        def _():
            gather_ref[:, pl.ds(me * k_shard, k_shard)] = (
                x_ref[pl.ds(me * m_blk, m_blk), :])

        for s in range(N_DEV):
            @pl.when(me != s)
            def _():
                rdma = pltpu.make_async_remote_copy(
                    src_ref=x_ref.at[pl.ds(s * m_blk, m_blk), :],
                    dst_ref=gather_ref.at[:, pl.ds(s * k_shard, k_shard)],
                    send_sem=send_sems.at[s],
                    recv_sem=recv_sems.at[s],
                    device_id=(s,),
                    device_id_type=pl.DeviceIdType.MESH,
                )
                rdma.wait_recv()
                rdma.wait_send()

        y = jnp.dot(gather_ref[:, :], w_ref[:, :],
                    preferred_element_type=jnp.float32)
        y = jnp.maximum(y, 0.0)
        y_ref[:, :] = y

        amax_ref[pl.ds(me, 1)] = jnp.full((1, 8, 128), jnp.max(y),
                                          dtype=jnp.float32)
        for j in range(N_DEV):
            @pl.when(me != j)
            def _():
                rdma = pltpu.make_async_remote_copy(
                    src_ref=amax_ref.at[pl.ds(me, 1)],
                    dst_ref=amax_ref.at[pl.ds(me, 1)],
                    send_sem=send_sems2.at[j],
                    recv_sem=recv_sems2.at[me],
                    device_id=(j,),
                    device_id_type=pl.DeviceIdType.MESH,
                )
                rdma.start()
        for s in range(N_DEV):
            @pl.when(me != s)
            def _():
                rdma = pltpu.make_async_remote_copy(
                    src_ref=amax_ref.at[pl.ds(s, 1)],
                    dst_ref=amax_ref.at[pl.ds(s, 1)],
                    send_sem=send_sems2.at[s],
                    recv_sem=recv_sems2.at[s],
                    device_id=(s,),
                    device_id_type=pl.DeviceIdType.MESH,
                )
                rdma.wait_recv()
                rdma.wait_send()

        gmax = jnp.max(amax_ref[:, :, :])
        scale = gmax / 448.0
        q = (y_ref[:, :] / scale).astype(jnp.float8_e4m3fn)
        out_ref[:, :] = q.astype(jnp.float32) * scale

    return pl.pallas_call(
        body,
        out_shape=jax.ShapeDtypeStruct((m_blk, n), jnp.float32),
        in_specs=[
            pl.BlockSpec(memory_space=pltpu.VMEM),
            pl.BlockSpec(memory_space=pltpu.VMEM),
        ],
        out_specs=pl.BlockSpec(memory_space=pltpu.VMEM),
        scratch_shapes=[
            pltpu.VMEM((m_blk, k_dim), jnp.float32),
            pltpu.VMEM((m_blk, n), jnp.float32),
            pltpu.VMEM((N_DEV, 8, 128), jnp.float32),
            pltpu.SemaphoreType.DMA((N_DEV,)),
            pltpu.SemaphoreType.DMA((N_DEV,)),
            pltpu.SemaphoreType.DMA((N_DEV,)),
            pltpu.SemaphoreType.DMA((N_DEV,)),
        ],
        compiler_params=pltpu.CompilerParams(collective_id=0),
    )(x, w_mat)


# baseline (device time: 83298 ns/iter reference)
import jax
import jax.numpy as jnp
from jax import lax
from jax.experimental import pallas as pl
from jax.experimental.pallas import tpu as pltpu

N_DEV = 16


def kernel(x, w_mat):
    k_dim, k_shard = x.shape
    n = w_mat.shape[1]
    m_blk = k_dim // N_DEV

    def body(x_ref, w_ref, out_ref, gather_ref, y_ref, amax_ref,
             send_sems, recv_sems, send_sems2, recv_sems2):
        me = lax.axis_index("i")

        barrier_sem = pltpu.get_barrier_semaphore()
        for d in range(N_DEV):
            @pl.when(me != d)
            def _():
                pl.semaphore_signal(
                    barrier_sem, inc=1,
                    device_id=(d,), device_id_type=pl.DeviceIdType.MESH,
                )
        pl.semaphore_wait(barrier_sem, N_DEV - 1)

        for j in range(N_DEV):
            @pl.when(me != j)
            def _():
                rdma = pltpu.make_async_remote_copy(
                    src_ref=x_ref.at[pl.ds(j * m_blk, m_blk), :],
                    dst_ref=gather_ref.at[:, pl.ds(me * k_shard, k_shard)],
                    send_sem=send_sems.at[j],
                    recv_sem=recv_sems.at[me],
                    device_id=(j,),
                    device_id_type=pl.DeviceIdType.MESH,
                )
                rdma.start()

        gather_ref[:, pl.ds(me * k_shard, k_shard)] = (
            x_ref[pl.ds(me * m_blk, m_blk), :])

        for s in range(N_DEV):
            @pl.when(me != s)
            def _():
                rdma = pltpu.make_async_remote_copy(
                    src_ref=x_ref.at[pl.ds(s * m_blk, m_blk), :],
                    dst_ref=gather_ref.at[:, pl.ds(s * k_shard, k_shard)],
                    send_sem=send_sems.at[s],
                    recv_sem=recv_sems.at[s],
                    device_id=(s,),
                    device_id_type=pl.DeviceIdType.MESH,
                )
                rdma.wait_recv()
                rdma.wait_send()

        y = jnp.dot(gather_ref[:, :], w_ref[:, :],
                    preferred_element_type=jnp.float32)
        y = jnp.maximum(y, 0.0)
        y_ref[:, :] = y

        amax_ref[pl.ds(me, 1)] = jnp.full((1, 8, 128), jnp.max(y),
                                          dtype=jnp.float32)
        for j in range(N_DEV):
            @pl.when(me != j)
            def _():
                rdma = pltpu.make_async_remote_copy(
                    src_ref=amax_ref.at[pl.ds(me, 1)],
                    dst_ref=amax_ref.at[pl.ds(me, 1)],
                    send_sem=send_sems2.at[j],
                    recv_sem=recv_sems2.at[me],
                    device_id=(j,),
                    device_id_type=pl.DeviceIdType.MESH,
                )
                rdma.start()
        for s in range(N_DEV):
            @pl.when(me != s)
            def _():
                rdma = pltpu.make_async_remote_copy(
                    src_ref=amax_ref.at[pl.ds(s, 1)],
                    dst_ref=amax_ref.at[pl.ds(s, 1)],
                    send_sem=send_sems2.at[s],
                    recv_sem=recv_sems2.at[s],
                    device_id=(s,),
                    device_id_type=pl.DeviceIdType.MESH,
                )
                rdma.wait_recv()
                rdma.wait_send()

        gmax = jnp.max(amax_ref[:, :, :])
        scale = gmax / 448.0
        q = (y_ref[:, :] / scale).astype(jnp.float8_e4m3fn)
        out_ref[:, :] = q.astype(jnp.float32) * scale

    return pl.pallas_call(
        body,
        out_shape=jax.ShapeDtypeStruct((m_blk, n), jnp.float32),
        in_specs=[
            pl.BlockSpec(memory_space=pltpu.VMEM),
            pl.BlockSpec(memory_space=pltpu.VMEM),
        ],
        out_specs=pl.BlockSpec(memory_space=pltpu.VMEM),
        scratch_shapes=[
            pltpu.VMEM((m_blk, k_dim), jnp.float32),
            pltpu.VMEM((m_blk, n), jnp.float32),
            pltpu.VMEM((N_DEV, 8, 128), jnp.float32),
            pltpu.SemaphoreType.DMA((N_DEV,)),
            pltpu.SemaphoreType.DMA((N_DEV,)),
            pltpu.SemaphoreType.DMA((N_DEV,)),
            pltpu.SemaphoreType.DMA((N_DEV,)),
        ],
        compiler_params=pltpu.CompilerParams(
            collective_id=0,
            vmem_limit_bytes=100 * 1024 * 1024,
        ),
    )(x, w_mat)
